# baseline (speedup 1.0000x reference)
import jax
import jax.numpy as jnp
from jax.experimental import pallas as pl
from jax.experimental.pallas import tpu as pltpu

_G = 64
_BB = 128


def _twohot_block(spec_ref, out_ref):
    s = spec_ref[...]
    sc = jnp.clip(s, 0.0, _G - 1.0 - 1e-06)
    lower = jnp.floor(sc)
    frac = sc - lower
    il = lower.astype(jnp.int32)[:, None, :]
    f = frac[:, None, :]
    g = jax.lax.broadcasted_iota(jnp.int32, out_ref.shape, 1)
    out_ref[...] = jnp.where(g == il, 1.0 - f, jnp.where(g == il + 1, f, 0.0))


def kernel(spec):
    b, d = spec.shape
    return pl.pallas_call(
        _twohot_block,
        grid=(b // _BB,),
        in_specs=[pl.BlockSpec((_BB, d), lambda i: (i, 0))],
        out_specs=pl.BlockSpec((_BB, _G, d), lambda i: (i, 0, 0)),
        out_shape=jax.ShapeDtypeStruct((b, _G, d), jnp.float32),
        compiler_params=pltpu.CompilerParams(
            dimension_semantics=("parallel",),
        ),
    )(spec)


# speedup vs baseline: 1.0245x; 1.0245x over previous
import jax
import jax.numpy as jnp
from jax.experimental import pallas as pl
from jax.experimental.pallas import tpu as pltpu

_G = 64
_BB = 128
_NBUF = 8


def _twohot_body(spec_ref, out_ref, scratch, sems):
    b = out_ref.shape[0]
    d = out_ref.shape[2]
    nchunk = b // _BB

    def chunk_copy(c, slot):
        return pltpu.make_async_copy(
            scratch.at[pl.ds(slot * _BB, _BB)],
            out_ref.at[pl.ds(c * _BB, _BB)],
            sems.at[slot],
        )

    def step(c, carry):
        slot = jax.lax.rem(c, _NBUF)

        @pl.when(c >= _NBUF)
        def _():
            chunk_copy(c - _NBUF, slot).wait()

        s = spec_ref[pl.ds(c * _BB, _BB), :]
        sc = jnp.clip(s, 0.0, _G - 1.0 - 1e-06)
        lower = jnp.floor(sc)
        frac = sc - lower
        il = lower.astype(jnp.int32)[:, None, :]
        f = frac[:, None, :]
        g = jax.lax.broadcasted_iota(jnp.int32, (_BB, _G, d), 1)
        scratch[pl.ds(slot * _BB, _BB)] = jnp.where(
            g == il, 1.0 - f, jnp.where(g == il + 1, f, 0.0)
        )
        chunk_copy(c, slot).start()
        return carry

    jax.lax.fori_loop(0, nchunk, step, 0)

    def drain(i, carry):
        c = nchunk - _NBUF + i
        chunk_copy(c, jax.lax.rem(c, _NBUF)).wait()
        return carry

    jax.lax.fori_loop(0, _NBUF, drain, 0)


def kernel(spec):
    b, d = spec.shape
    return pl.pallas_call(
        _twohot_body,
        in_specs=[pl.BlockSpec(memory_space=pltpu.MemorySpace.VMEM)],
        out_specs=pl.BlockSpec(memory_space=pl.ANY),
        out_shape=jax.ShapeDtypeStruct((b, _G, d), jnp.float32),
        scratch_shapes=[
            pltpu.VMEM((_NBUF * _BB, _G, d), jnp.float32),
            pltpu.SemaphoreType.DMA((_NBUF,)),
        ],
    )(spec)
